# baseline (speedup 1.0000x reference)
import jax
import jax.numpy as jnp
import numpy as np
from jax import lax
from jax.experimental import pallas as pl
from jax.experimental.pallas import tpu as pltpu
from jax.experimental.pallas import tpu_sc as plsc

N = 10000
D = 128
E = 320000
B = 10000

NC, NS, L = 2, 16, 16
NW = NC * NS
N_PAD = 10240
ROWS_PER_TILE = N_PAD // NS
E_PAD = NW * 80 * 128
EG = 80
B_PAD = NW * 3 * 128
BG = 3

G0 = 120
G1 = 40

_MESH = plsc.VectorSubcoreMesh(
    core_axis_name="c", subcore_axis_name="s", num_cores=NC, num_subcores=NS
)
_SC_PARAMS = pltpu.CompilerParams(use_tc_tiling_on_sc=False,
                                  needs_layout_passes=False)


def _edge_body(src_hbm, dst_hbm, feat_hbm, agg_hbm, deg_hbm,
               srcb, dstb, gb0, gb1, rf0, rf1, ones, zblk, semg, sems, semd,
               agg_sh, deg_sh):
    cid = lax.axis_index("c")
    sid = lax.axis_index("s")
    wid = sid * NC + cid
    row0 = sid * ROWS_PER_TILE

    zf = jnp.zeros((L,), jnp.float32)
    of = jnp.ones((L,), jnp.float32)
    msk = jnp.full((L,), -65536, jnp.int32)
    sh16 = jnp.full((L,), 16, jnp.int32)

    def _zrows(i, _):
        r = i // 8
        c = (i % 8) * L
        rf0[r, pl.ds(c, L)] = zf
        rf1[r, pl.ds(c, L)] = zf
        return 0
    lax.fori_loop(0, 64 * 8, _zrows, 0)

    def _zblk(i, _):
        ones[i, :] = of
        zblk[i, :] = zf
        return 0
    lax.fori_loop(0, 64, _zblk, 0)

    for k in range(ROWS_PER_TILE // 64):
        pltpu.sync_copy(rf0, agg_sh.at[pl.ds(row0 + k * 64, 64)])
        pltpu.sync_copy(zblk, deg_sh.at[pl.ds(row0 + k * 64, 64)])
    plsc.subcore_barrier()

    n_groups = jnp.where(cid == 0, G0, G1)
    gbase = sid * (G0 + G1) + cid * G0

    pltpu.sync_copy(src_hbm.at[pl.ds(gbase, 8)], srcb)
    pltpu.sync_copy(dst_hbm.at[pl.ds(2 * gbase, 16)], dstb)

    def _convert(gc, h, rf):
        @plsc.parallel_loop(0, 64, unroll=4)
        def _r(r):
            for c in range(4):
                x = gc[h * 64 + r, pl.ds(c * L, L)]
                lo = plsc.bitcast(lax.shift_left(x, sh16), jnp.float32)
                hi = plsc.bitcast(x, jnp.float32)
                rf[r, pl.ds(c * 32, L)] = lo
                rf[r, pl.ds(c * 32 + L, L)] = hi

    def _stage(t, _):
        pltpu.async_copy(feat_hbm.at[srcb.at[0]], gb0, semg)
        for j in range(8):
            gc = gb0 if j % 2 == 0 else gb1
            gn = gb1 if j % 2 == 0 else gb0
            if j < 7:
                pltpu.async_copy(feat_hbm.at[srcb.at[j + 1]], gn, semg)
            pltpu.make_async_copy(feat_hbm.at[srcb.at[j]], gc, semg).wait()
            for h in range(2):
                rf = rf0 if h == 0 else rf1
                if j >= 1:
                    pltpu.make_async_copy(
                        rf, agg_sh.at[dstb.at[2 * j + h - 2]], sems).wait()
                _convert(gc, h, rf)
                pltpu.async_copy(
                    rf, agg_sh.at[dstb.at[2 * j + h]], sems, add=True)
                pltpu.async_copy(
                    ones, deg_sh.at[dstb.at[2 * j + h]], semd, add=True)
        pltpu.make_async_copy(rf0, agg_sh.at[dstb.at[14]], sems).wait()
        pltpu.make_async_copy(rf1, agg_sh.at[dstb.at[15]], sems).wait()
        for j in range(16):
            pltpu.make_async_copy(ones, deg_sh.at[dstb.at[j]], semd).wait()

        @pl.when(t < n_groups // 8 - 1)
        def _():
            pltpu.sync_copy(src_hbm.at[pl.ds(gbase + (t + 1) * 8, 8)], srcb)
            pltpu.sync_copy(
                dst_hbm.at[pl.ds(2 * gbase + (t + 1) * 16, 16)], dstb)
        return 0
    lax.fori_loop(0, n_groups // 8, _stage, 0)

    plsc.subcore_barrier()

    out0 = cid * N_PAD + row0
    pltpu.sync_copy(agg_sh.at[pl.ds(row0, ROWS_PER_TILE)],
                    agg_hbm.at[pl.ds(out0, ROWS_PER_TILE)])
    pltpu.sync_copy(deg_sh.at[pl.ds(row0, ROWS_PER_TILE)],
                    deg_hbm.at[pl.ds(out0, ROWS_PER_TILE)])


_edge_kernel = pl.kernel(
    _edge_body,
    out_type=(
        jax.ShapeDtypeStruct((NC * N_PAD, D), jnp.float32),
        jax.ShapeDtypeStruct((NC * N_PAD, L), jnp.float32),
    ),
    mesh=_MESH,
    scratch_types=[
        pltpu.VMEM((8, 128), jnp.int32),
        pltpu.VMEM((16, 64), jnp.int32),
        pltpu.VMEM((128, D // 2), jnp.int32),
        pltpu.VMEM((128, D // 2), jnp.int32),
        pltpu.VMEM((64, D), jnp.float32),
        pltpu.VMEM((64, D), jnp.float32),
        pltpu.VMEM((64, L), jnp.float32),
        pltpu.VMEM((64, L), jnp.float32),
        pltpu.SemaphoreType.DMA,
        pltpu.SemaphoreType.DMA,
        pltpu.SemaphoreType.DMA,
        pltpu.VMEM_SHARED((N_PAD, D), jnp.float32),
        pltpu.VMEM_SHARED((N_PAD, L), jnp.float32),
    ],
    compiler_params=_SC_PARAMS,
)

_ORIG = np.empty((D,), np.int32)
for _c in range(4):
    for _j in range(16):
        _ORIG[32 * _c + _j] = 32 * _c + 2 * _j
        _ORIG[32 * _c + 16 + _j] = 32 * _c + 2 * _j + 1


def _dense_body(feat, a0, a1, d0, d1, w, b, p_out, q_out, r_out):
    w1a = w[0:D, :]
    w1b = w[D:2 * D, :]
    p_out[...] = jnp.dot(feat[...], w1a, preferred_element_type=jnp.float32) + b[...]
    q_out[...] = jnp.dot(a0[...] + a1[...], w1b, preferred_element_type=jnp.float32)
    r_out[...] = 1.0 / jnp.maximum(d0[...] + d1[...], 1.0)


_DENSE_R = 1280

_dense_kernel = pl.pallas_call(
    _dense_body,
    grid=(N_PAD // _DENSE_R,),
    in_specs=[
        pl.BlockSpec((_DENSE_R, D), lambda i: (i, 0)),
        pl.BlockSpec((_DENSE_R, D), lambda i: (i, 0)),
        pl.BlockSpec((_DENSE_R, D), lambda i: (i + N_PAD // _DENSE_R, 0)),
        pl.BlockSpec((_DENSE_R, L), lambda i: (i, 0)),
        pl.BlockSpec((_DENSE_R, L), lambda i: (i + N_PAD // _DENSE_R, 0)),
        pl.BlockSpec((2 * D, D), lambda i: (0, 0)),
        pl.BlockSpec((1, D), lambda i: (0, 0)),
    ],
    out_specs=[
        pl.BlockSpec((_DENSE_R, D), lambda i: (i, 0)),
        pl.BlockSpec((_DENSE_R, D), lambda i: (i, 0)),
        pl.BlockSpec((_DENSE_R, L), lambda i: (i, 0)),
    ],
    out_shape=[
        jax.ShapeDtypeStruct((N_PAD, D), jnp.float32),
        jax.ShapeDtypeStruct((N_PAD, D), jnp.float32),
        jax.ShapeDtypeStruct((N_PAD, L), jnp.float32),
    ],
)


def _gather_body(p_hbm, q_hbm, r_hbm, nidx_hbm, out_hbm,
                 nib, pb0, pb1, pb2, qb0, qb1, qb2, rb, semg, semw):
    cid = lax.axis_index("c")
    sid = lax.axis_index("s")
    wid = sid * NC + cid
    pbs = (pb0, pb1, pb2)
    qbs = (qb0, qb1, qb2)

    with jax.named_scope("bg_idx"):
        pltpu.sync_copy(nidx_hbm.at[wid], nib)

    with jax.named_scope("bg_fire"):
        for g in range(BG):
            pltpu.async_copy(p_hbm.at[nib.at[g]], pbs[g], semg)
            pltpu.async_copy(q_hbm.at[nib.at[g]], qbs[g], semg)
            pltpu.async_copy(r_hbm.at[nib.at[g]], rb.at[g], semg)

    for g in range(BG):
        with jax.named_scope(f"bg_wait{g}"):
            pltpu.make_async_copy(p_hbm.at[nib.at[g]], pbs[g], semg).wait()
            pltpu.make_async_copy(q_hbm.at[nib.at[g]], qbs[g], semg).wait()
            pltpu.make_async_copy(r_hbm.at[nib.at[g]], rb.at[g], semg).wait()
        pb = pbs[g]
        qb = qbs[g]

        with jax.named_scope(f"bg_comp{g}"):
            def _rows(r, _):
                rinv = rb[g, r, :]
                for j in range(D // L):
                    s = pl.ds(j * L, L)
                    pb[r, s] = jnp.maximum(pb[r, s] + qb[r, s] * rinv, 0.0)
                return 0
            lax.fori_loop(0, 128, _rows, 0)

            base = wid * (BG * 128) + g * 128
            pltpu.async_copy(pb, out_hbm.at[pl.ds(base, 128)], semw)
    with jax.named_scope("bg_drain"):
        for g in range(BG):
            pltpu.make_async_copy(pbs[g], out_hbm.at[pl.ds(0, 128)], semw).wait()


_gather_kernel = pl.kernel(
    _gather_body,
    out_type=jax.ShapeDtypeStruct((B_PAD, D), jnp.float32),
    mesh=_MESH,
    scratch_types=[
        pltpu.VMEM((BG, 128), jnp.int32),
        pltpu.VMEM((128, D), jnp.float32),
        pltpu.VMEM((128, D), jnp.float32),
        pltpu.VMEM((128, D), jnp.float32),
        pltpu.VMEM((128, D), jnp.float32),
        pltpu.VMEM((128, D), jnp.float32),
        pltpu.VMEM((128, D), jnp.float32),
        pltpu.VMEM((BG, 128, L), jnp.float32),
        pltpu.SemaphoreType.DMA,
        pltpu.SemaphoreType.DMA,
    ],
    compiler_params=_SC_PARAMS,
)


@jax.jit
def kernel(nodes, edge_index, feat_table, W1, b1):
    src = edge_index[0].astype(jnp.int32)
    dst = edge_index[1].astype(jnp.int32)
    src_p = jnp.concatenate(
        [src, jnp.zeros((E_PAD - E,), jnp.int32)]).reshape(E_PAD // 128, 128)
    pad_dst = N + jax.lax.rem(jnp.arange(E_PAD - E, dtype=jnp.int32),
                              jnp.int32(N_PAD - N))
    dst_p = jnp.concatenate([dst, pad_dst]).reshape(E_PAD // 64, 64)
    feat_pairs = jax.lax.bitcast_convert_type(
        feat_table.astype(jnp.bfloat16).reshape(N, D // 2, 2), jnp.int32)
    agg, deg = _edge_kernel(src_p, dst_p, feat_pairs)
    w1_perm = jnp.concatenate([W1[:D], W1[D:][_ORIG]], axis=0)
    p, q, r = _dense_kernel(feat_table, agg, agg, deg, deg, w1_perm,
                            b1.reshape(1, D))
    nodes_p = jnp.concatenate(
        [nodes.astype(jnp.int32), jnp.zeros((B_PAD - B,), jnp.int32)]
    ).reshape(NW, BG, 128)
    outp = _gather_kernel(p, q, r, nodes_p)
    return outp[:B]


# speedup vs baseline: 1.0321x; 1.0321x over previous
import jax
import jax.numpy as jnp
import numpy as np
from jax import lax
from jax.experimental import pallas as pl
from jax.experimental.pallas import tpu as pltpu
from jax.experimental.pallas import tpu_sc as plsc

N = 10000
D = 128
E = 320000
B = 10000

NC, NS, L = 2, 16, 16
NW = NC * NS
N_PAD = 10240
ROWS_PER_TILE = N_PAD // NS
E_PAD = NW * 80 * 128
EG = 80
B_PAD = NW * 3 * 128
BG = 3

G0 = 112
G1 = 48

_MESH = plsc.VectorSubcoreMesh(
    core_axis_name="c", subcore_axis_name="s", num_cores=NC, num_subcores=NS
)
_SC_PARAMS = pltpu.CompilerParams(use_tc_tiling_on_sc=False,
                                  needs_layout_passes=False)


def _edge_body(src_hbm, dst_hbm, feat_hbm, agg_hbm, deg_hbm,
               srcb, dstb, gb0, gb1, rf0, rf1, ones, zblk, semg, sems, semd,
               agg_sh, deg_sh):
    cid = lax.axis_index("c")
    sid = lax.axis_index("s")
    wid = sid * NC + cid
    row0 = sid * ROWS_PER_TILE

    zf = jnp.zeros((L,), jnp.float32)
    of = jnp.ones((L,), jnp.float32)
    msk = jnp.full((L,), -65536, jnp.int32)
    sh16 = jnp.full((L,), 16, jnp.int32)

    def _zrows(i, _):
        r = i // 8
        c = (i % 8) * L
        rf0[r, pl.ds(c, L)] = zf
        rf1[r, pl.ds(c, L)] = zf
        return 0
    lax.fori_loop(0, 64 * 8, _zrows, 0)

    def _zblk(i, _):
        ones[i, :] = of
        zblk[i, :] = zf
        return 0
    lax.fori_loop(0, 64, _zblk, 0)

    for k in range(ROWS_PER_TILE // 64):
        pltpu.sync_copy(rf0, agg_sh.at[pl.ds(row0 + k * 64, 64)])
        pltpu.sync_copy(zblk, deg_sh.at[pl.ds(row0 + k * 64, 64)])
    plsc.subcore_barrier()

    n_groups = jnp.where(cid == 0, G0, G1)
    gbase = sid * (G0 + G1) + cid * G0

    pltpu.sync_copy(src_hbm.at[pl.ds(gbase, 8)], srcb)
    pltpu.sync_copy(dst_hbm.at[pl.ds(2 * gbase, 16)], dstb)

    def _convert(gc, h, rf):
        @plsc.parallel_loop(0, 64, unroll=4)
        def _r(r):
            for c in range(4):
                x = gc[h * 64 + r, pl.ds(c * L, L)]
                lo = plsc.bitcast(lax.shift_left(x, sh16), jnp.float32)
                hi = plsc.bitcast(x, jnp.float32)
                rf[r, pl.ds(c * 32, L)] = lo
                rf[r, pl.ds(c * 32 + L, L)] = hi

    def _stage(t, _):
        pltpu.async_copy(feat_hbm.at[srcb.at[0]], gb0, semg)
        for j in range(8):
            gc = gb0 if j % 2 == 0 else gb1
            gn = gb1 if j % 2 == 0 else gb0
            if j < 7:
                pltpu.async_copy(feat_hbm.at[srcb.at[j + 1]], gn, semg)
            pltpu.make_async_copy(feat_hbm.at[srcb.at[j]], gc, semg).wait()
            for h in range(2):
                rf = rf0 if h == 0 else rf1
                if j >= 1:
                    pltpu.make_async_copy(
                        rf, agg_sh.at[dstb.at[2 * j + h - 2]], sems).wait()
                _convert(gc, h, rf)
                pltpu.async_copy(
                    rf, agg_sh.at[dstb.at[2 * j + h]], sems, add=True)
                pltpu.async_copy(
                    ones, deg_sh.at[dstb.at[2 * j + h]], semd, add=True)
        pltpu.make_async_copy(rf0, agg_sh.at[dstb.at[14]], sems).wait()
        pltpu.make_async_copy(rf1, agg_sh.at[dstb.at[15]], sems).wait()
        for j in range(16):
            pltpu.make_async_copy(ones, deg_sh.at[dstb.at[j]], semd).wait()

        @pl.when(t < n_groups // 8 - 1)
        def _():
            pltpu.sync_copy(src_hbm.at[pl.ds(gbase + (t + 1) * 8, 8)], srcb)
            pltpu.sync_copy(
                dst_hbm.at[pl.ds(2 * gbase + (t + 1) * 16, 16)], dstb)
        return 0
    lax.fori_loop(0, n_groups // 8, _stage, 0)

    plsc.subcore_barrier()

    out0 = cid * N_PAD + row0
    pltpu.sync_copy(agg_sh.at[pl.ds(row0, ROWS_PER_TILE)],
                    agg_hbm.at[pl.ds(out0, ROWS_PER_TILE)])
    pltpu.sync_copy(deg_sh.at[pl.ds(row0, ROWS_PER_TILE)],
                    deg_hbm.at[pl.ds(out0, ROWS_PER_TILE)])


_edge_kernel = pl.kernel(
    _edge_body,
    out_type=(
        jax.ShapeDtypeStruct((NC * N_PAD, D), jnp.float32),
        jax.ShapeDtypeStruct((NC * N_PAD, L), jnp.float32),
    ),
    mesh=_MESH,
    scratch_types=[
        pltpu.VMEM((8, 128), jnp.int32),
        pltpu.VMEM((16, 64), jnp.int32),
        pltpu.VMEM((128, D // 2), jnp.int32),
        pltpu.VMEM((128, D // 2), jnp.int32),
        pltpu.VMEM((64, D), jnp.float32),
        pltpu.VMEM((64, D), jnp.float32),
        pltpu.VMEM((64, L), jnp.float32),
        pltpu.VMEM((64, L), jnp.float32),
        pltpu.SemaphoreType.DMA,
        pltpu.SemaphoreType.DMA,
        pltpu.SemaphoreType.DMA,
        pltpu.VMEM_SHARED((N_PAD, D), jnp.float32),
        pltpu.VMEM_SHARED((N_PAD, L), jnp.float32),
    ],
    compiler_params=_SC_PARAMS,
)

_ORIG = np.empty((D,), np.int32)
for _c in range(4):
    for _j in range(16):
        _ORIG[32 * _c + _j] = 32 * _c + 2 * _j
        _ORIG[32 * _c + 16 + _j] = 32 * _c + 2 * _j + 1


def _dense_body(feat, a0, a1, d0, d1, w, b, p_out, q_out, r_out):
    w1a = w[0:D, :]
    w1b = w[D:2 * D, :]
    p_out[...] = jnp.dot(feat[...], w1a, preferred_element_type=jnp.float32) + b[...]
    q_out[...] = jnp.dot(a0[...] + a1[...], w1b, preferred_element_type=jnp.float32)
    r_out[...] = 1.0 / jnp.maximum(d0[...] + d1[...], 1.0)


_DENSE_R = 1280

_dense_kernel = pl.pallas_call(
    _dense_body,
    grid=(N_PAD // _DENSE_R,),
    in_specs=[
        pl.BlockSpec((_DENSE_R, D), lambda i: (i, 0)),
        pl.BlockSpec((_DENSE_R, D), lambda i: (i, 0)),
        pl.BlockSpec((_DENSE_R, D), lambda i: (i + N_PAD // _DENSE_R, 0)),
        pl.BlockSpec((_DENSE_R, L), lambda i: (i, 0)),
        pl.BlockSpec((_DENSE_R, L), lambda i: (i + N_PAD // _DENSE_R, 0)),
        pl.BlockSpec((2 * D, D), lambda i: (0, 0)),
        pl.BlockSpec((1, D), lambda i: (0, 0)),
    ],
    out_specs=[
        pl.BlockSpec((_DENSE_R, D), lambda i: (i, 0)),
        pl.BlockSpec((_DENSE_R, D), lambda i: (i, 0)),
        pl.BlockSpec((_DENSE_R, L), lambda i: (i, 0)),
    ],
    out_shape=[
        jax.ShapeDtypeStruct((N_PAD, D), jnp.float32),
        jax.ShapeDtypeStruct((N_PAD, D), jnp.float32),
        jax.ShapeDtypeStruct((N_PAD, L), jnp.float32),
    ],
)


def _gather_body(p_hbm, q_hbm, r_hbm, nidx_hbm, out_hbm,
                 nib, pb0, pb1, pb2, qb0, qb1, qb2, rb, semg, semw):
    cid = lax.axis_index("c")
    sid = lax.axis_index("s")
    wid = sid * NC + cid
    pbs = (pb0, pb1, pb2)
    qbs = (qb0, qb1, qb2)

    with jax.named_scope("bg_idx"):
        pltpu.sync_copy(nidx_hbm.at[wid], nib)

    with jax.named_scope("bg_fire"):
        for g in range(BG):
            pltpu.async_copy(p_hbm.at[nib.at[g]], pbs[g], semg)
            pltpu.async_copy(q_hbm.at[nib.at[g]], qbs[g], semg)
            pltpu.async_copy(r_hbm.at[nib.at[g]], rb.at[g], semg)

    for g in range(BG):
        with jax.named_scope(f"bg_wait{g}"):
            pltpu.make_async_copy(p_hbm.at[nib.at[g]], pbs[g], semg).wait()
            pltpu.make_async_copy(q_hbm.at[nib.at[g]], qbs[g], semg).wait()
            pltpu.make_async_copy(r_hbm.at[nib.at[g]], rb.at[g], semg).wait()
        pb = pbs[g]
        qb = qbs[g]

        with jax.named_scope(f"bg_comp{g}"):
            def _rows(r, _):
                rinv = rb[g, r, :]
                for j in range(D // L):
                    s = pl.ds(j * L, L)
                    pb[r, s] = jnp.maximum(pb[r, s] + qb[r, s] * rinv, 0.0)
                return 0
            lax.fori_loop(0, 128, _rows, 0)

            base = wid * (BG * 128) + g * 128
            pltpu.async_copy(pb, out_hbm.at[pl.ds(base, 128)], semw)
    with jax.named_scope("bg_drain"):
        for g in range(BG):
            pltpu.make_async_copy(pbs[g], out_hbm.at[pl.ds(0, 128)], semw).wait()


_gather_kernel = pl.kernel(
    _gather_body,
    out_type=jax.ShapeDtypeStruct((B_PAD, D), jnp.float32),
    mesh=_MESH,
    scratch_types=[
        pltpu.VMEM((BG, 128), jnp.int32),
        pltpu.VMEM((128, D), jnp.float32),
        pltpu.VMEM((128, D), jnp.float32),
        pltpu.VMEM((128, D), jnp.float32),
        pltpu.VMEM((128, D), jnp.float32),
        pltpu.VMEM((128, D), jnp.float32),
        pltpu.VMEM((128, D), jnp.float32),
        pltpu.VMEM((BG, 128, L), jnp.float32),
        pltpu.SemaphoreType.DMA,
        pltpu.SemaphoreType.DMA,
    ],
    compiler_params=_SC_PARAMS,
)


@jax.jit
def kernel(nodes, edge_index, feat_table, W1, b1):
    src = edge_index[0].astype(jnp.int32)
    dst = edge_index[1].astype(jnp.int32)
    src_p = jnp.concatenate(
        [src, jnp.zeros((E_PAD - E,), jnp.int32)]).reshape(E_PAD // 128, 128)
    pad_dst = N + jax.lax.rem(jnp.arange(E_PAD - E, dtype=jnp.int32),
                              jnp.int32(N_PAD - N))
    dst_p = jnp.concatenate([dst, pad_dst]).reshape(E_PAD // 64, 64)
    feat_pairs = jax.lax.bitcast_convert_type(
        feat_table.astype(jnp.bfloat16).reshape(N, D // 2, 2), jnp.int32)
    agg, deg = _edge_kernel(src_p, dst_p, feat_pairs)
    w1_perm = jnp.concatenate([W1[:D], W1[D:][_ORIG]], axis=0)
    p, q, r = _dense_kernel(feat_table, agg, agg, deg, deg, w1_perm,
                            b1.reshape(1, D))
    nodes_p = jnp.concatenate(
        [nodes.astype(jnp.int32), jnp.zeros((B_PAD - B,), jnp.int32)]
    ).reshape(NW, BG, 128)
    outp = _gather_kernel(p, q, r, nodes_p)
    return outp[:B]
